# baseline (speedup 1.0000x reference)
import jax
import jax.numpy as jnp
from jax import lax
from jax.experimental import pallas as pl
from jax.experimental.pallas import tpu as pltpu
from jax.experimental.pallas import tpu_sc as plsc

N = 10000
E = 320000
DIN = 128
DE = 16
DOUT = 128
H = 4
HD = 32
N2 = 10240
CE = 40
NTILES = 32
EP = E // NTILES


def _head_sum_matrix():
    r = lax.broadcasted_iota(jnp.int32, (DOUT, H), 0) // HD
    c = lax.broadcasted_iota(jnp.int32, (DOUT, H), 1)
    return (r == c).astype(jnp.float32)


def _k1_body(nf_ref, w1_ref, b_ref, af_ref, p_ref, sap_ref, gma_ref):
    i = pl.program_id(0)
    p = jnp.dot(nf_ref[...], w1_ref[...], preferred_element_type=jnp.float32)
    p = p + b_ref[...]
    p_ref[...] = p
    sap = jnp.dot(p * af_ref[...], _head_sum_matrix(),
                  preferred_element_type=jnp.float32)
    sap_ref[...] = sap
    bmax = jnp.max(sap, axis=0, keepdims=True)

    @pl.when(i == 0)
    def _():
        gma_ref[...] = bmax

    @pl.when(i > 0)
    def _():
        gma_ref[...] = jnp.maximum(gma_ref[...], bmax)


def _k2a_body(ef_ref, w2_ref, af_ref, sb_ref, gmb_ref):
    i = pl.program_id(0)
    v = jnp.dot(w2_ref[...] * af_ref[...], _head_sum_matrix(),
                preferred_element_type=jnp.float32)
    sb = jnp.dot(ef_ref[...], v, preferred_element_type=jnp.float32)
    sb_ref[...] = sb
    bmax = jnp.max(sb, axis=0, keepdims=True)

    @pl.when(i == 0)
    def _():
        gmb_ref[...] = bmax

    @pl.when(i > 0)
    def _():
        gmb_ref[...] = jnp.maximum(gmb_ref[...], bmax)


def _k2b_body(ef_ref, w2_ref, q_ref):
    q_ref[...] = jnp.dot(ef_ref[...], w2_ref[...],
                         preferred_element_type=jnp.float32)


def _k4_body(ss_ref, si_ref):
    sm = ss_ref[...]
    si_ref[...] = jnp.where(sm > 0.0, 1.0 / sm, 0.0)


def _kex_body(sa_ref, sb_ref, gb_ref, ex_ref):
    ex_ref[...] = jnp.exp(sa_ref[...] + sb_ref[...] - gb_ref[...])


def _kt2_body(q_ref, ex_ref, si_ref, a_ref, t2_ref):
    r = lax.broadcasted_iota(jnp.int32, (H, DOUT), 0)
    cc = lax.broadcasted_iota(jnp.int32, (H, DOUT), 1) // HD
    gt = (r == cc).astype(jnp.float32)
    alpha = ex_ref[...] * si_ref[...]
    alpha_exp = jnp.dot(alpha, gt, preferred_element_type=jnp.float32)
    a_ref[...] = alpha_exp
    t2_ref[...] = q_ref[...] * alpha_exp


def _sc_agg(p, aexp, t2, srce, dste, acc_out,
            src_v, dst_v, p_v, a_v, t2_v, msg_v, zb_v, acc_sh, sem):
    c = lax.axis_index("c")
    s = lax.axis_index("s")
    wid = s * 2 + c
    for j in range(16):
        for k in range(DOUT // 16):
            zb_v[j, pl.ds(k * 16, 16)] = jnp.zeros((16,), jnp.float32)

    def zr(r2, carry):
        pltpu.sync_copy(zb_v, acc_sh.at[pl.ds(s * 640 + r2 * 16, 16), :])
        return carry
    lax.fori_loop(0, 40, zr, 0)
    plsc.subcore_barrier()

    def chunk(i, carry):
        base = wid * EP + i * CE
        pltpu.sync_copy(srce.at[pl.ds(base, CE)], src_v)
        pltpu.sync_copy(dste.at[pl.ds(base, CE)], dst_v)
        pltpu.async_copy(p.at[src_v], p_v, sem).wait()
        pltpu.sync_copy(aexp.at[pl.ds(base, CE), :], a_v)
        pltpu.sync_copy(t2.at[pl.ds(base, CE), :], t2_v)
        for e in range(CE):
            for k in range(DOUT // 16):
                w = pl.ds(k * 16, 16)
                msg_v[e, w] = p_v[e, w] * a_v[e, w] + t2_v[e, w]
        pltpu.sync_copy(msg_v, acc_sh.at[dst_v], add=True)
        return carry
    lax.fori_loop(0, EP // CE, chunk, 0)

    plsc.subcore_barrier()
    pltpu.sync_copy(acc_sh.at[pl.ds(s * 640, 640), :],
                    acc_out.at[c, pl.ds(s * 640, 640), :])


def _k6_body(acc_ref, nf_ref, w_ref, b_ref, g_ref, bb_ref, o_ref):
    hn = acc_ref[0] + acc_ref[1] + nf_ref[...]
    z = jnp.dot(hn, w_ref[...], preferred_element_type=jnp.float32)
    z = z + b_ref[...]
    h = jnp.maximum(z, 0.0)
    mu = jnp.mean(h, axis=-1, keepdims=True)
    vr = jnp.mean((h - mu) ** 2, axis=-1, keepdims=True)
    o_ref[...] = (h - mu) * lax.rsqrt(vr + 1e-5) * g_ref[...] + bb_ref[...]


def kernel(nfeats, efeats, edge_index, W_proj_w, W_proj_b, attn_vec,
           W_out_w, W_out_b, ln_gamma, ln_beta):
    f32 = jnp.float32
    src_idx = edge_index[0]
    dst_idx = edge_index[1]
    W1 = W_proj_w[:DIN]
    W2 = W_proj_w[DIN:]
    af = attn_vec.reshape(1, DOUT)
    bias = W_proj_b.reshape(1, DOUT)

    nblk = 2000
    k1 = pl.pallas_call(
        _k1_body,
        grid=(N // nblk,),
        in_specs=[
            pl.BlockSpec((nblk, DIN), lambda i: (i, 0)),
            pl.BlockSpec((DIN, DOUT), lambda i: (0, 0)),
            pl.BlockSpec((1, DOUT), lambda i: (0, 0)),
            pl.BlockSpec((1, DOUT), lambda i: (0, 0)),
        ],
        out_specs=[
            pl.BlockSpec((nblk, DOUT), lambda i: (i, 0)),
            pl.BlockSpec((nblk, H), lambda i: (i, 0)),
            pl.BlockSpec((1, H), lambda i: (0, 0)),
        ],
        out_shape=[
            jax.ShapeDtypeStruct((N, DOUT), f32),
            jax.ShapeDtypeStruct((N, H), f32),
            jax.ShapeDtypeStruct((1, H), f32),
        ],
    )
    P, sap, gma = k1(nfeats, W1, bias, af)

    eblk = 4000
    k2a = pl.pallas_call(
        _k2a_body,
        grid=(E // eblk,),
        in_specs=[
            pl.BlockSpec((eblk, DE), lambda i: (i, 0)),
            pl.BlockSpec((DE, DOUT), lambda i: (0, 0)),
            pl.BlockSpec((1, DOUT), lambda i: (0, 0)),
        ],
        out_specs=[
            pl.BlockSpec((eblk, H), lambda i: (i, 0)),
            pl.BlockSpec((1, H), lambda i: (0, 0)),
        ],
        out_shape=[
            jax.ShapeDtypeStruct((E, H), f32),
            jax.ShapeDtypeStruct((1, H), f32),
        ],
    )
    sb, gmb = k2a(efeats, W2, af)

    k2b = pl.pallas_call(
        _k2b_body,
        grid=(E // eblk,),
        in_specs=[
            pl.BlockSpec((eblk, DE), lambda i: (i, 0)),
            pl.BlockSpec((DE, DOUT), lambda i: (0, 0)),
        ],
        out_specs=pl.BlockSpec((eblk, DOUT), lambda i: (i, 0)),
        out_shape=jax.ShapeDtypeStruct((E, DOUT), f32),
    )
    q = k2b(efeats, W2)

    gb = (gma + gmb).reshape(1, H)

    sa_g = jnp.take(sap, src_idx, axis=0)
    kex = pl.pallas_call(
        _kex_body,
        grid=(E // eblk,),
        in_specs=[
            pl.BlockSpec((eblk, H), lambda i: (i, 0)),
            pl.BlockSpec((eblk, H), lambda i: (i, 0)),
            pl.BlockSpec((1, H), lambda i: (0, 0)),
        ],
        out_specs=pl.BlockSpec((eblk, H), lambda i: (i, 0)),
        out_shape=jax.ShapeDtypeStruct((E, H), f32),
    )
    ex = kex(sa_g, sb, gb)
    ssum = jax.ops.segment_sum(ex, dst_idx, num_segments=N)

    k4 = pl.pallas_call(
        _k4_body,
        grid=(5,),
        in_specs=[pl.BlockSpec((N // 5, H), lambda i: (i, 0))],
        out_specs=pl.BlockSpec((N // 5, H), lambda i: (i, 0)),
        out_shape=jax.ShapeDtypeStruct((N, H), f32),
    )
    sinv = k4(ssum)

    si_g = jnp.take(sinv, dst_idx, axis=0)
    kt2 = pl.pallas_call(
        _kt2_body,
        grid=(E // eblk,),
        in_specs=[
            pl.BlockSpec((eblk, DOUT), lambda i: (i, 0)),
            pl.BlockSpec((eblk, H), lambda i: (i, 0)),
            pl.BlockSpec((eblk, H), lambda i: (i, 0)),
        ],
        out_specs=[
            pl.BlockSpec((eblk, DOUT), lambda i: (i, 0)),
            pl.BlockSpec((eblk, DOUT), lambda i: (i, 0)),
        ],
        out_shape=[
            jax.ShapeDtypeStruct((E, DOUT), f32),
            jax.ShapeDtypeStruct((E, DOUT), f32),
        ],
    )
    aexp, t2 = kt2(q, ex, si_g)
    scmesh = plsc.VectorSubcoreMesh(core_axis_name="c", subcore_axis_name="s")
    ksc = pl.kernel(
        _sc_agg,
        mesh=scmesh,
        out_type=jax.ShapeDtypeStruct((2, N2, DOUT), f32),
        scratch_types=[
            pltpu.VMEM((CE,), jnp.int32),
            pltpu.VMEM((CE,), jnp.int32),
            pltpu.VMEM((CE, DOUT), f32),
            pltpu.VMEM((CE, DOUT), f32),
            pltpu.VMEM((CE, DOUT), f32),
            pltpu.VMEM((CE, DOUT), f32),
            pltpu.VMEM((16, DOUT), f32),
            pltpu.VMEM_SHARED((N2, DOUT), f32),
            pltpu.SemaphoreType.DMA,
        ],
    )
    acc = ksc(P, aexp, t2, src_idx, dst_idx)

    k6 = pl.pallas_call(
        _k6_body,
        grid=(N // nblk,),
        in_specs=[
            pl.BlockSpec((2, nblk, DOUT), lambda i: (0, i, 0)),
            pl.BlockSpec((nblk, DOUT), lambda i: (i, 0)),
            pl.BlockSpec((DOUT, DOUT), lambda i: (0, 0)),
            pl.BlockSpec((1, DOUT), lambda i: (0, 0)),
            pl.BlockSpec((1, DOUT), lambda i: (0, 0)),
            pl.BlockSpec((1, DOUT), lambda i: (0, 0)),
        ],
        out_specs=pl.BlockSpec((nblk, DOUT), lambda i: (i, 0)),
        out_shape=jax.ShapeDtypeStruct((N, DOUT), f32),
    )
    out = k6(acc, nfeats, W_out_w, W_out_b.reshape(1, DOUT),
             ln_gamma.reshape(1, DOUT), ln_beta.reshape(1, DOUT))
    return out


# speedup vs baseline: 1.0221x; 1.0221x over previous
import jax
import jax.numpy as jnp
from jax import lax
from jax.experimental import pallas as pl
from jax.experimental.pallas import tpu as pltpu
from jax.experimental.pallas import tpu_sc as plsc

N = 10000
E = 320000
DIN = 128
DE = 16
DOUT = 128
H = 4
HD = 32
N2 = 10240
CE = 40
NTILES = 32
EP = E // NTILES


def _head_sum_matrix():
    r = lax.broadcasted_iota(jnp.int32, (DOUT, H), 0) // HD
    c = lax.broadcasted_iota(jnp.int32, (DOUT, H), 1)
    return (r == c).astype(jnp.float32)


def _k1_body(nf_ref, w1_ref, b_ref, af_ref, p_ref, sap_ref, gma_ref):
    i = pl.program_id(0)
    p = jnp.dot(nf_ref[...], w1_ref[...], preferred_element_type=jnp.float32)
    p = p + b_ref[...]
    p_ref[...] = p
    sap = jnp.dot(p * af_ref[...], _head_sum_matrix(),
                  preferred_element_type=jnp.float32)
    sap_ref[...] = sap
    bmax = jnp.max(sap, axis=0, keepdims=True)

    @pl.when(i == 0)
    def _():
        gma_ref[...] = bmax

    @pl.when(i > 0)
    def _():
        gma_ref[...] = jnp.maximum(gma_ref[...], bmax)


def _k2a_body(ef_ref, w2_ref, af_ref, sb_ref, gmb_ref):
    i = pl.program_id(0)
    v = jnp.dot(w2_ref[...] * af_ref[...], _head_sum_matrix(),
                preferred_element_type=jnp.float32)
    sb = jnp.dot(ef_ref[...], v, preferred_element_type=jnp.float32)
    sb_ref[...] = sb
    bmax = jnp.max(sb, axis=0, keepdims=True)

    @pl.when(i == 0)
    def _():
        gmb_ref[...] = bmax

    @pl.when(i > 0)
    def _():
        gmb_ref[...] = jnp.maximum(gmb_ref[...], bmax)


def _k2b_body(ef_ref, w2_ref, q_ref):
    q_ref[...] = jnp.dot(ef_ref[...], w2_ref[...],
                         preferred_element_type=jnp.float32)


def _k4_body(ss_ref, si_ref):
    sm = ss_ref[...]
    si_ref[...] = jnp.where(sm > 0.0, 1.0 / sm, 0.0)


def _kex_body(sa_ref, sb_ref, gb_ref, ex_ref):
    ex_ref[...] = jnp.exp(sa_ref[...] + sb_ref[...] - gb_ref[...])


def _kt2_body(ex_ref, si_ref, a_ref):
    r = lax.broadcasted_iota(jnp.int32, (H, DOUT), 0)
    cc = lax.broadcasted_iota(jnp.int32, (H, DOUT), 1) // HD
    gt = (r == cc).astype(jnp.float32)
    alpha = ex_ref[...] * si_ref[...]
    a_ref[...] = jnp.dot(alpha, gt, preferred_element_type=jnp.float32)


def _sc_agg(p, aexp, q, srce, dste, acc_out,
            src_v, dst_v, p_v, a_v, q_v, msg_v, zb_v, acc_sh, sem):
    c = lax.axis_index("c")
    s = lax.axis_index("s")
    wid = s * 2 + c
    for j in range(16):
        for k in range(DOUT // 16):
            zb_v[j, pl.ds(k * 16, 16)] = jnp.zeros((16,), jnp.float32)

    def zr(r2, carry):
        pltpu.sync_copy(zb_v, acc_sh.at[pl.ds(s * 640 + r2 * 16, 16), :])
        return carry
    lax.fori_loop(0, 40, zr, 0)
    plsc.subcore_barrier()

    def chunk(i, carry):
        base = wid * EP + i * CE
        pltpu.sync_copy(srce.at[pl.ds(base, CE)], src_v)
        pltpu.sync_copy(dste.at[pl.ds(base, CE)], dst_v)
        pltpu.async_copy(p.at[src_v], p_v, sem).wait()
        pltpu.sync_copy(aexp.at[pl.ds(base, CE), :], a_v)
        pltpu.sync_copy(q.at[pl.ds(base, CE), :], q_v)
        for e in range(CE):
            for k in range(DOUT // 16):
                w = pl.ds(k * 16, 16)
                msg_v[e, w] = (p_v[e, w] + q_v[e, w]) * a_v[e, w]
        pltpu.sync_copy(msg_v, acc_sh.at[dst_v], add=True)
        return carry
    lax.fori_loop(0, EP // CE, chunk, 0)

    plsc.subcore_barrier()
    pltpu.sync_copy(acc_sh.at[pl.ds(s * 640, 640), :],
                    acc_out.at[c, pl.ds(s * 640, 640), :])


def _k6_body(acc_ref, nf_ref, w_ref, b_ref, g_ref, bb_ref, o_ref):
    hn = acc_ref[0] + acc_ref[1] + nf_ref[...]
    z = jnp.dot(hn, w_ref[...], preferred_element_type=jnp.float32)
    z = z + b_ref[...]
    h = jnp.maximum(z, 0.0)
    mu = jnp.mean(h, axis=-1, keepdims=True)
    vr = jnp.mean((h - mu) ** 2, axis=-1, keepdims=True)
    o_ref[...] = (h - mu) * lax.rsqrt(vr + 1e-5) * g_ref[...] + bb_ref[...]


def kernel(nfeats, efeats, edge_index, W_proj_w, W_proj_b, attn_vec,
           W_out_w, W_out_b, ln_gamma, ln_beta):
    f32 = jnp.float32
    src_idx = edge_index[0]
    dst_idx = edge_index[1]
    W1 = W_proj_w[:DIN]
    W2 = W_proj_w[DIN:]
    af = attn_vec.reshape(1, DOUT)
    bias = W_proj_b.reshape(1, DOUT)

    nblk = 2000
    k1 = pl.pallas_call(
        _k1_body,
        grid=(N // nblk,),
        in_specs=[
            pl.BlockSpec((nblk, DIN), lambda i: (i, 0)),
            pl.BlockSpec((DIN, DOUT), lambda i: (0, 0)),
            pl.BlockSpec((1, DOUT), lambda i: (0, 0)),
            pl.BlockSpec((1, DOUT), lambda i: (0, 0)),
        ],
        out_specs=[
            pl.BlockSpec((nblk, DOUT), lambda i: (i, 0)),
            pl.BlockSpec((nblk, H), lambda i: (i, 0)),
            pl.BlockSpec((1, H), lambda i: (0, 0)),
        ],
        out_shape=[
            jax.ShapeDtypeStruct((N, DOUT), f32),
            jax.ShapeDtypeStruct((N, H), f32),
            jax.ShapeDtypeStruct((1, H), f32),
        ],
    )
    P, sap, gma = k1(nfeats, W1, bias, af)

    eblk = 4000
    k2a = pl.pallas_call(
        _k2a_body,
        grid=(E // eblk,),
        in_specs=[
            pl.BlockSpec((eblk, DE), lambda i: (i, 0)),
            pl.BlockSpec((DE, DOUT), lambda i: (0, 0)),
            pl.BlockSpec((1, DOUT), lambda i: (0, 0)),
        ],
        out_specs=[
            pl.BlockSpec((eblk, H), lambda i: (i, 0)),
            pl.BlockSpec((1, H), lambda i: (0, 0)),
        ],
        out_shape=[
            jax.ShapeDtypeStruct((E, H), f32),
            jax.ShapeDtypeStruct((1, H), f32),
        ],
    )
    sb, gmb = k2a(efeats, W2, af)

    k2b = pl.pallas_call(
        _k2b_body,
        grid=(E // eblk,),
        in_specs=[
            pl.BlockSpec((eblk, DE), lambda i: (i, 0)),
            pl.BlockSpec((DE, DOUT), lambda i: (0, 0)),
        ],
        out_specs=pl.BlockSpec((eblk, DOUT), lambda i: (i, 0)),
        out_shape=jax.ShapeDtypeStruct((E, DOUT), f32),
    )
    q = k2b(efeats, W2)

    gb = (gma + gmb).reshape(1, H)

    sa_g = jnp.take(sap, src_idx, axis=0)
    kex = pl.pallas_call(
        _kex_body,
        grid=(E // eblk,),
        in_specs=[
            pl.BlockSpec((eblk, H), lambda i: (i, 0)),
            pl.BlockSpec((eblk, H), lambda i: (i, 0)),
            pl.BlockSpec((1, H), lambda i: (0, 0)),
        ],
        out_specs=pl.BlockSpec((eblk, H), lambda i: (i, 0)),
        out_shape=jax.ShapeDtypeStruct((E, H), f32),
    )
    ex = kex(sa_g, sb, gb)
    ssum = jax.ops.segment_sum(ex, dst_idx, num_segments=N)

    k4 = pl.pallas_call(
        _k4_body,
        grid=(5,),
        in_specs=[pl.BlockSpec((N // 5, H), lambda i: (i, 0))],
        out_specs=pl.BlockSpec((N // 5, H), lambda i: (i, 0)),
        out_shape=jax.ShapeDtypeStruct((N, H), f32),
    )
    sinv = k4(ssum)

    si_g = jnp.take(sinv, dst_idx, axis=0)
    kt2 = pl.pallas_call(
        _kt2_body,
        grid=(E // eblk,),
        in_specs=[
            pl.BlockSpec((eblk, H), lambda i: (i, 0)),
            pl.BlockSpec((eblk, H), lambda i: (i, 0)),
        ],
        out_specs=pl.BlockSpec((eblk, DOUT), lambda i: (i, 0)),
        out_shape=jax.ShapeDtypeStruct((E, DOUT), f32),
    )
    aexp = kt2(ex, si_g)
    scmesh = plsc.VectorSubcoreMesh(core_axis_name="c", subcore_axis_name="s")
    ksc = pl.kernel(
        _sc_agg,
        mesh=scmesh,
        out_type=jax.ShapeDtypeStruct((2, N2, DOUT), f32),
        scratch_types=[
            pltpu.VMEM((CE,), jnp.int32),
            pltpu.VMEM((CE,), jnp.int32),
            pltpu.VMEM((CE, DOUT), f32),
            pltpu.VMEM((CE, DOUT), f32),
            pltpu.VMEM((CE, DOUT), f32),
            pltpu.VMEM((CE, DOUT), f32),
            pltpu.VMEM((16, DOUT), f32),
            pltpu.VMEM_SHARED((N2, DOUT), f32),
            pltpu.SemaphoreType.DMA,
        ],
    )
    acc = ksc(P, aexp, q, src_idx, dst_idx)

    k6 = pl.pallas_call(
        _k6_body,
        grid=(N // nblk,),
        in_specs=[
            pl.BlockSpec((2, nblk, DOUT), lambda i: (0, i, 0)),
            pl.BlockSpec((nblk, DOUT), lambda i: (i, 0)),
            pl.BlockSpec((DOUT, DOUT), lambda i: (0, 0)),
            pl.BlockSpec((1, DOUT), lambda i: (0, 0)),
            pl.BlockSpec((1, DOUT), lambda i: (0, 0)),
            pl.BlockSpec((1, DOUT), lambda i: (0, 0)),
        ],
        out_specs=pl.BlockSpec((nblk, DOUT), lambda i: (i, 0)),
        out_shape=jax.ShapeDtypeStruct((N, DOUT), f32),
    )
    out = k6(acc, nfeats, W_out_w, W_out_b.reshape(1, DOUT),
             ln_gamma.reshape(1, DOUT), ln_beta.reshape(1, DOUT))
    return out
